# Initial kernel scaffold; baseline (speedup 1.0000x reference)
import functools

import jax
import jax.numpy as jnp
from jax import lax
from jax.experimental import pallas as pl
from jax.experimental.pallas import tpu as pltpu
from jax.experimental.pallas import tpu_sc as plsc

H = 4
C = 128

NC = 2
NS = 16
LANES = 16

CH = 80
ROWW = 144
ZR = 125


def _tc_nodes_body(x_ref, w_ref, asrc_ref, adst_ref, h_ref, as_ref, ad_ref):
    hb = jnp.dot(x_ref[...], w_ref[...], preferred_element_type=jnp.float32)
    as_cols = []
    ad_cols = []
    for hd in range(H):
        sl = slice(hd * C, (hd + 1) * C)
        h_ref[hd] = hb[:, sl]
        as_cols.append(jnp.sum(hb[:, sl] * asrc_ref[0:1, sl], axis=1, keepdims=True))
        ad_cols.append(jnp.sum(hb[:, sl] * adst_ref[0:1, sl], axis=1, keepdims=True))
    as_ref[...] = jnp.concatenate(as_cols, axis=1)
    ad_ref[...] = jnp.concatenate(ad_cols, axis=1)


def _tc_nodes(x, W, att_src_flat, att_dst_flat):
    n = x.shape[0]
    bn = 500
    grid = (n // bn,)
    return pl.pallas_call(
        _tc_nodes_body,
        grid=grid,
        in_specs=[
            pl.BlockSpec((bn, x.shape[1]), lambda i: (i, 0)),
            pl.BlockSpec(W.shape, lambda i: (0, 0)),
            pl.BlockSpec((1, H * C), lambda i: (0, 0)),
            pl.BlockSpec((1, H * C), lambda i: (0, 0)),
        ],
        out_specs=[
            pl.BlockSpec((H, bn, C), lambda i: (0, i, 0)),
            pl.BlockSpec((bn, H), lambda i: (i, 0)),
            pl.BlockSpec((bn, H), lambda i: (i, 0)),
        ],
        out_shape=[
            jax.ShapeDtypeStruct((H, n, C), jnp.float32),
            jax.ShapeDtypeStruct((n, H), jnp.float32),
            jax.ShapeDtypeStruct((n, H), jnp.float32),
        ],
    )(x, W, att_src_flat, att_dst_flat)


def _tc_edges_body(ea_ref, lw_ref, atte_ref, ae_ref, ael_ref):
    ve_cols = []
    for hd in range(H):
        sl = slice(hd * C, (hd + 1) * C)
        ve_cols.append(jnp.sum(lw_ref[:, sl] * atte_ref[0:1, sl], axis=1, keepdims=True))
    ve = jnp.concatenate(ve_cols, axis=1)
    aeb = jnp.dot(ea_ref[...], ve, preferred_element_type=jnp.float32)
    ae_ref[...] = aeb

    @pl.when(pl.program_id(0) == 0)
    def _():
        ael_ref[...] = jnp.zeros_like(ael_ref)

    ael_ref[...] += jnp.sum(aeb, axis=0)[:, None]


def _tc_edges(edge_attr, lin_edge_W, att_edge_flat):
    E = edge_attr.shape[0]
    be = 512
    grid = (E // be,)
    return pl.pallas_call(
        _tc_edges_body,
        grid=grid,
        in_specs=[
            pl.BlockSpec((be, edge_attr.shape[1]), lambda i: (i, 0)),
            pl.BlockSpec(lin_edge_W.shape, lambda i: (0, 0)),
            pl.BlockSpec((1, H * C), lambda i: (0, 0)),
        ],
        out_specs=[
            pl.BlockSpec((be, H), lambda i: (i, 0)),
            pl.BlockSpec((H, 128), lambda i: (0, 0)),
        ],
        out_shape=[
            jax.ShapeDtypeStruct((E, H), jnp.float32),
            jax.ShapeDtypeStruct((H, 128), jnp.float32),
        ],
    )(edge_attr, lin_edge_W, att_edge_flat)


def _sc_body(n, E, h_hbm, asT_hbm, adT_hbm, aeT_hbm, src_hbm, dst_hbm, out_hbm,
             as_v, ad_v, srcb, dstb, gidx, aeb, wb, hrows, msg, zb, accsh, sem):
    cid = lax.axis_index("c")
    sid = lax.axis_index("s")
    epw = E // NS
    base_e = sid * epw
    npw = n // NS
    iota = lax.iota(jnp.int32, LANES)
    zvec = jnp.zeros((LANES,), jnp.float32)

    def _z_zb(i, carry):
        r = i // (ROWW // LANES)
        c = (i % (ROWW // LANES)) * LANES
        zb[r, pl.ds(c, LANES)] = zvec
        return carry

    lax.fori_loop(0, ZR * (ROWW // LANES), _z_zb, 0)

    def _z_msg(e, carry):
        msg[e, pl.ds(C, LANES)] = zvec
        return carry

    lax.fori_loop(0, CH, _z_msg, 0)

    for k in range(2):
        head = cid * 2 + k

        pltpu.sync_copy(asT_hbm.at[pl.ds(head * n, n)], as_v)
        pltpu.sync_copy(adT_hbm.at[pl.ds(head * n, n)], ad_v)

        for z in range(npw // ZR):
            pltpu.sync_copy(zb, accsh.at[pl.ds(sid * npw + z * ZR, ZR)])
        plsc.subcore_barrier()

        def _chunk(ci, carry):
            eb = base_e + ci * CH
            pltpu.sync_copy(src_hbm.at[pl.ds(eb, CH)], srcb)
            pltpu.sync_copy(dst_hbm.at[pl.ds(eb, CH)], dstb)
            pltpu.sync_copy(aeT_hbm.at[pl.ds(head * E + eb, CH)], aeb)

            for g in range(CH // LANES):
                s16 = srcb[pl.ds(g * LANES, LANES)]
                d16 = dstb[pl.ds(g * LANES, LANES)]
                gidx[pl.ds(g * LANES, LANES)] = s16 + head * n
                av = plsc.load_gather(as_v, [s16])
                dv = plsc.load_gather(ad_v, [d16])
                lg = av + dv + aeb[pl.ds(g * LANES, LANES)]
                lg = jnp.maximum(lg, 0.2 * lg)
                wb[pl.ds(g * LANES, LANES)] = jnp.exp(lg)

            pltpu.async_copy(h_hbm.at[gidx], hrows, sem).wait()

            def _edge(e, carry2):
                ws = wb[e]
                for j in range(C // LANES):
                    msg[e, pl.ds(j * LANES, LANES)] = hrows[e, pl.ds(j * LANES, LANES)] * ws
                msg[e, pl.ds(C, LANES)] = jnp.where(iota == 0, ws, 0.0)
                return carry2

            lax.fori_loop(0, CH, _edge, 0)

            pltpu.sync_copy(msg, accsh.at[dstb], add=True)
            return carry

        lax.fori_loop(0, epw // CH, _chunk, 0)
        plsc.subcore_barrier()

        pltpu.sync_copy(accsh.at[pl.ds(sid * npw, npw)],
                        out_hbm.at[pl.ds(head * n + sid * npw, npw)])
        plsc.subcore_barrier()


def _sc_aggregate(h_flat, asT, adT, aeT, src, dst, n, E):
    mesh = plsc.VectorSubcoreMesh(core_axis_name="c", subcore_axis_name="s",
                                  num_cores=NC, num_subcores=NS)
    fn = pl.kernel(
        functools.partial(_sc_body, n, E),
        out_type=jax.ShapeDtypeStruct((H * n, ROWW), jnp.float32),
        mesh=mesh,
        scratch_types=[
            pltpu.VMEM((n,), jnp.float32),
            pltpu.VMEM((n,), jnp.float32),
            pltpu.VMEM((CH,), jnp.int32),
            pltpu.VMEM((CH,), jnp.int32),
            pltpu.VMEM((CH,), jnp.int32),
            pltpu.VMEM((CH,), jnp.float32),
            pltpu.VMEM((CH,), jnp.float32),
            pltpu.VMEM((CH, C), jnp.float32),
            pltpu.VMEM((CH, ROWW), jnp.float32),
            pltpu.VMEM((ZR, ROWW), jnp.float32),
            pltpu.VMEM_SHARED((10000, ROWW), jnp.float32),
            pltpu.SemaphoreType.DMA,
        ],
    )
    return fn(h_flat, asT, adT, aeT, src, dst)


def _tc_final_body(inv_e, acc_ref, h_ref, as_ref, ad_ref, ael_ref, bias_ref, out_ref):
    for hd in range(H):
        sl = slice(hd * C, (hd + 1) * C)
        ael = ael_ref[hd : hd + 1, 0:1] * inv_e
        lg = as_ref[:, hd : hd + 1] + ad_ref[:, hd : hd + 1] + ael
        lg = jnp.maximum(lg, 0.2 * lg)
        wl = jnp.exp(lg)
        num = acc_ref[hd, :, 0:C] + wl * h_ref[hd]
        den = acc_ref[hd, :, C : C + 1] + wl
        out_ref[:, sl] = num / den + bias_ref[0:1, sl]


def _tc_final(acc, h, a_src, a_dst, ael, bias_flat, E):
    n = h.shape[1]
    bn = 500
    grid = (n // bn,)
    return pl.pallas_call(
        functools.partial(_tc_final_body, 1.0 / E),
        grid=grid,
        in_specs=[
            pl.BlockSpec((H, bn, ROWW), lambda i: (0, i, 0)),
            pl.BlockSpec((H, bn, C), lambda i: (0, i, 0)),
            pl.BlockSpec((bn, H), lambda i: (i, 0)),
            pl.BlockSpec((bn, H), lambda i: (i, 0)),
            pl.BlockSpec((H, 128), lambda i: (0, 0)),
            pl.BlockSpec((1, H * C), lambda i: (0, 0)),
        ],
        out_specs=pl.BlockSpec((bn, H * C), lambda i: (i, 0)),
        out_shape=jax.ShapeDtypeStruct((n, H * C), jnp.float32),
    )(acc, h, a_src, a_dst, ael, bias_flat)


def kernel(x, edge_index, edge_attr, W, att_src, att_dst, lin_edge_W, att_edge, bias):
    n = x.shape[0]
    E = edge_index.shape[1]
    src = edge_index[0]
    dst = edge_index[1]

    h, a_src, a_dst = _tc_nodes(x, W, att_src.reshape(1, H * C), att_dst.reshape(1, H * C))
    a_e, ael = _tc_edges(edge_attr, lin_edge_W, att_edge.reshape(1, H * C))

    h_flat = h.reshape(H * n, C)
    asT = a_src.T.reshape(H * n)
    adT = a_dst.T.reshape(H * n)
    aeT = a_e.T.reshape(H * E)

    acc = _sc_aggregate(h_flat, asT, adT, aeT, src, dst, n, E)

    out = _tc_final(acc.reshape(H, n, ROWW), h, a_src, a_dst, ael,
                    bias.reshape(1, H * C), E)
    return out


# speedup vs baseline: 9.5553x; 9.5553x over previous
import functools

import jax
import jax.numpy as jnp
from jax import lax
from jax.experimental import pallas as pl
from jax.experimental.pallas import tpu as pltpu
from jax.experimental.pallas import tpu_sc as plsc

H = 4
C = 128

NC = 2
NS = 16
LANES = 16

CH = 80
NPAD = 10240


def _tc_nodes_body(x_ref, w_ref, asrc_ref, adst_ref, h_ref, as_ref, ad_ref):
    hb = jnp.dot(x_ref[...], w_ref[...], preferred_element_type=jnp.float32)
    as_cols = []
    ad_cols = []
    for hd in range(H):
        sl = slice(hd * C, (hd + 1) * C)
        h_ref[hd] = hb[:, sl]
        as_cols.append(jnp.sum(hb[:, sl] * asrc_ref[0:1, sl], axis=1, keepdims=True))
        ad_cols.append(jnp.sum(hb[:, sl] * adst_ref[0:1, sl], axis=1, keepdims=True))
    as_ref[...] = jnp.concatenate(as_cols, axis=1)
    ad_ref[...] = jnp.concatenate(ad_cols, axis=1)


def _tc_nodes(x, W, att_src_flat, att_dst_flat):
    n = x.shape[0]
    bn = 1000
    grid = (n // bn,)
    return pl.pallas_call(
        _tc_nodes_body,
        grid=grid,
        in_specs=[
            pl.BlockSpec((bn, x.shape[1]), lambda i: (i, 0)),
            pl.BlockSpec(W.shape, lambda i: (0, 0)),
            pl.BlockSpec((1, H * C), lambda i: (0, 0)),
            pl.BlockSpec((1, H * C), lambda i: (0, 0)),
        ],
        out_specs=[
            pl.BlockSpec((H, bn, C), lambda i: (0, i, 0)),
            pl.BlockSpec((bn, H), lambda i: (i, 0)),
            pl.BlockSpec((bn, H), lambda i: (i, 0)),
        ],
        out_shape=[
            jax.ShapeDtypeStruct((H, n, C), jnp.float32),
            jax.ShapeDtypeStruct((n, H), jnp.float32),
            jax.ShapeDtypeStruct((n, H), jnp.float32),
        ],
    )(x, W, att_src_flat, att_dst_flat)


def _tc_edges_body(ea_ref, lw_ref, atte_ref, ae_ref, ael_ref):
    ve_cols = []
    for hd in range(H):
        sl = slice(hd * C, (hd + 1) * C)
        ve_cols.append(jnp.sum(lw_ref[:, sl] * atte_ref[0:1, sl], axis=1, keepdims=True))
    ve = jnp.concatenate(ve_cols, axis=1)
    aeb = jnp.dot(ea_ref[...], ve, preferred_element_type=jnp.float32)
    ae_ref[...] = aeb

    @pl.when(pl.program_id(0) == 0)
    def _():
        ael_ref[...] = jnp.zeros_like(ael_ref)

    ael_ref[...] += jnp.sum(aeb, axis=0)[:, None]


def _tc_edges(edge_attr, lin_edge_W, att_edge_flat):
    E = edge_attr.shape[0]
    be = 512
    grid = (E // be,)
    return pl.pallas_call(
        _tc_edges_body,
        grid=grid,
        in_specs=[
            pl.BlockSpec((be, edge_attr.shape[1]), lambda i: (i, 0)),
            pl.BlockSpec(lin_edge_W.shape, lambda i: (0, 0)),
            pl.BlockSpec((1, H * C), lambda i: (0, 0)),
        ],
        out_specs=[
            pl.BlockSpec((be, H), lambda i: (i, 0)),
            pl.BlockSpec((H, 128), lambda i: (0, 0)),
        ],
        out_shape=[
            jax.ShapeDtypeStruct((E, H), jnp.float32),
            jax.ShapeDtypeStruct((H, 128), jnp.float32),
        ],
    )(edge_attr, lin_edge_W, att_edge_flat)


def _sc_body(n, E, h_hbm, asT_hbm, adT_hbm, aeT_hbm, src_hbm, dst_hbm,
             out_hbm, w_hbm,
             as_v, ad_v, srcb, dstb, gidx, aeb, wb, hrows, msg, accsh, sem):
    cid = lax.axis_index("c")
    sid = lax.axis_index("s")
    epw = E // NS
    base_e = sid * epw
    npw = NPAD // NS

    for k in range(2):
        head = cid * 2 + k

        pltpu.sync_copy(asT_hbm.at[pl.ds(head * n, n)], as_v)
        pltpu.sync_copy(adT_hbm.at[pl.ds(head * n, n)], ad_v)

        def _z_msg(i, carry):
            r = i // (C // LANES)
            c = (i % (C // LANES)) * LANES
            msg[r, pl.ds(c, LANES)] = jnp.zeros((LANES,), jnp.float32)
            return carry

        lax.fori_loop(0, CH * (C // LANES), _z_msg, 0)
        for z in range(npw // CH):
            pltpu.sync_copy(msg, accsh.at[pl.ds(sid * npw + z * CH, CH)])
        plsc.subcore_barrier()

        def _chunk(ci, carry):
            eb = base_e + ci * CH
            pltpu.sync_copy(src_hbm.at[pl.ds(eb, CH)], srcb)
            pltpu.sync_copy(dst_hbm.at[pl.ds(eb, CH)], dstb)
            pltpu.sync_copy(aeT_hbm.at[pl.ds(head * E + eb, CH)], aeb)

            for g in range(CH // LANES):
                s16 = srcb[pl.ds(g * LANES, LANES)]
                d16 = dstb[pl.ds(g * LANES, LANES)]
                gidx[pl.ds(g * LANES, LANES)] = s16 + head * n
                av = plsc.load_gather(as_v, [s16])
                dv = plsc.load_gather(ad_v, [d16])
                lg = av + dv + aeb[pl.ds(g * LANES, LANES)]
                lg = jnp.maximum(lg, 0.2 * lg)
                wb[pl.ds(g * LANES, LANES)] = jnp.exp(lg)

            pltpu.sync_copy(wb, w_hbm.at[pl.ds(head * E + eb, CH)])
            pltpu.async_copy(h_hbm.at[gidx], hrows, sem).wait()

            def _egrp(g, carry2):
                w16 = wb[pl.ds(g * LANES, LANES)]
                for lane in range(LANES):
                    e = g * LANES + lane
                    ws = w16[lane]
                    for j in range(C // LANES):
                        msg[e, pl.ds(j * LANES, LANES)] = hrows[e, pl.ds(j * LANES, LANES)] * ws
                return carry2

            lax.fori_loop(0, CH // LANES, _egrp, 0)

            pltpu.sync_copy(msg, accsh.at[dstb], add=True)
            return carry

        lax.fori_loop(0, epw // CH, _chunk, 0)
        plsc.subcore_barrier()

        pltpu.sync_copy(accsh.at[pl.ds(sid * npw, npw)],
                        out_hbm.at[pl.ds(head * NPAD + sid * npw, npw)])
        plsc.subcore_barrier()


def _sc_aggregate(h_flat, asT, adT, aeT, src, dst, n, E):
    mesh = plsc.VectorSubcoreMesh(core_axis_name="c", subcore_axis_name="s",
                                  num_cores=NC, num_subcores=NS)
    fn = pl.kernel(
        functools.partial(_sc_body, n, E),
        out_type=(
            jax.ShapeDtypeStruct((H * NPAD, C), jnp.float32),
            jax.ShapeDtypeStruct((H * E,), jnp.float32),
        ),
        mesh=mesh,
        compiler_params=pltpu.CompilerParams(needs_layout_passes=False),
        scratch_types=[
            pltpu.VMEM((n,), jnp.float32),
            pltpu.VMEM((n,), jnp.float32),
            pltpu.VMEM((CH,), jnp.int32),
            pltpu.VMEM((CH,), jnp.int32),
            pltpu.VMEM((CH,), jnp.int32),
            pltpu.VMEM((CH,), jnp.float32),
            pltpu.VMEM((CH,), jnp.float32),
            pltpu.VMEM((CH, C), jnp.float32),
            pltpu.VMEM((CH, C), jnp.float32),
            pltpu.VMEM_SHARED((NPAD, C), jnp.float32),
            pltpu.SemaphoreType.DMA,
        ],
    )
    return fn(h_flat, asT, adT, aeT, src, dst)


def _tc_s_body(dst_ref, w_ref, s_ref):
    d = dst_ref[...]
    q = lax.shift_right_logical(d, 7)
    r = jnp.bitwise_and(d, 127)
    be = d.shape[0]
    oq = (q == lax.broadcasted_iota(jnp.int32, (be, NPAD // 128), 1)).astype(jnp.float32)
    orr = (r == lax.broadcasted_iota(jnp.int32, (be, 128), 1)).astype(jnp.float32)

    @pl.when(pl.program_id(0) == 0)
    def _():
        s_ref[...] = jnp.zeros_like(s_ref)

    for hd in range(H):
        a = oq * w_ref[:, hd : hd + 1]
        s_ref[hd] += lax.dot_general(a, orr, (((0,), (0,)), ((), ())),
                                     preferred_element_type=jnp.float32)


def _tc_s(dst2d, wT):
    E = dst2d.shape[0]
    be = 512
    grid = (E // be,)
    return pl.pallas_call(
        _tc_s_body,
        grid=grid,
        in_specs=[
            pl.BlockSpec((be, 1), lambda i: (i, 0)),
            pl.BlockSpec((be, H), lambda i: (i, 0)),
        ],
        out_specs=pl.BlockSpec((H, NPAD // 128, 128), lambda i: (0, 0, 0)),
        out_shape=jax.ShapeDtypeStruct((H, NPAD // 128, 128), jnp.float32),
    )(dst2d, wT)


def _tc_final_body(inv_e, acc_ref, h_ref, as_ref, ad_ref, s_ref, ael_ref,
                   bias_ref, out_ref):
    for hd in range(H):
        sl = slice(hd * C, (hd + 1) * C)
        ael = ael_ref[hd : hd + 1, 0:1] * inv_e
        lg = as_ref[:, hd : hd + 1] + ad_ref[:, hd : hd + 1] + ael
        lg = jnp.maximum(lg, 0.2 * lg)
        wl = jnp.exp(lg)
        num = acc_ref[hd] + wl * h_ref[hd]
        den = s_ref[:, hd : hd + 1] + wl
        out_ref[:, sl] = num / den + bias_ref[0:1, sl]


def _tc_final(acc, h, a_src, a_dst, s_nodes, ael, bias_flat, E):
    n = h.shape[1]
    bn = 1000
    grid = (n // bn,)
    return pl.pallas_call(
        functools.partial(_tc_final_body, 1.0 / E),
        grid=grid,
        in_specs=[
            pl.BlockSpec((H, bn, C), lambda i: (0, i, 0)),
            pl.BlockSpec((H, bn, C), lambda i: (0, i, 0)),
            pl.BlockSpec((bn, H), lambda i: (i, 0)),
            pl.BlockSpec((bn, H), lambda i: (i, 0)),
            pl.BlockSpec((bn, H), lambda i: (i, 0)),
            pl.BlockSpec((H, 128), lambda i: (0, 0)),
            pl.BlockSpec((1, H * C), lambda i: (0, 0)),
        ],
        out_specs=pl.BlockSpec((bn, H * C), lambda i: (i, 0)),
        out_shape=jax.ShapeDtypeStruct((n, H * C), jnp.float32),
    )(acc, h, a_src, a_dst, s_nodes, ael, bias_flat)


def kernel(x, edge_index, edge_attr, W, att_src, att_dst, lin_edge_W, att_edge, bias):
    n = x.shape[0]
    E = edge_index.shape[1]
    src = edge_index[0]
    dst = edge_index[1]

    h, a_src, a_dst = _tc_nodes(x, W, att_src.reshape(1, H * C), att_dst.reshape(1, H * C))
    a_e, ael = _tc_edges(edge_attr, lin_edge_W, att_edge.reshape(1, H * C))

    h_flat = h.reshape(H * n, C)
    asT = a_src.T.reshape(H * n)
    adT = a_dst.T.reshape(H * n)
    aeT = a_e.T.reshape(H * E)

    acc, w_planes = _sc_aggregate(h_flat, asT, adT, aeT, src, dst, n, E)

    s = _tc_s(dst.reshape(E, 1), w_planes.reshape(H, E).T)
    s_nodes = s.reshape(H, NPAD).T

    out = _tc_final(acc.reshape(H, NPAD, C), h, a_src, a_dst, s_nodes, ael,
                    bias.reshape(1, H * C), E)
    return out


# speedup vs baseline: 14.9728x; 1.5670x over previous
import functools

import jax
import jax.numpy as jnp
from jax import lax
from jax.experimental import pallas as pl
from jax.experimental.pallas import tpu as pltpu
from jax.experimental.pallas import tpu_sc as plsc

H = 4
C = 128

NC = 2
NS = 16
LANES = 16

CH = 80
NPAD = 10240


def _tc_nodes_body(x_ref, w_ref, asrc_ref, adst_ref, h_ref, as_ref, ad_ref):
    hb = jnp.dot(x_ref[...], w_ref[...], preferred_element_type=jnp.float32)
    as_cols = []
    ad_cols = []
    for hd in range(H):
        sl = slice(hd * C, (hd + 1) * C)
        h_ref[hd] = hb[:, sl]
        as_cols.append(jnp.sum(hb[:, sl] * asrc_ref[0:1, sl], axis=1, keepdims=True))
        ad_cols.append(jnp.sum(hb[:, sl] * adst_ref[0:1, sl], axis=1, keepdims=True))
    as_ref[...] = jnp.concatenate(as_cols, axis=1)
    ad_ref[...] = jnp.concatenate(ad_cols, axis=1)


def _tc_nodes(x, W, att_src_flat, att_dst_flat):
    n = x.shape[0]
    bn = 1000
    grid = (n // bn,)
    return pl.pallas_call(
        _tc_nodes_body,
        grid=grid,
        in_specs=[
            pl.BlockSpec((bn, x.shape[1]), lambda i: (i, 0)),
            pl.BlockSpec(W.shape, lambda i: (0, 0)),
            pl.BlockSpec((1, H * C), lambda i: (0, 0)),
            pl.BlockSpec((1, H * C), lambda i: (0, 0)),
        ],
        out_specs=[
            pl.BlockSpec((H, bn, C), lambda i: (0, i, 0)),
            pl.BlockSpec((bn, H), lambda i: (i, 0)),
            pl.BlockSpec((bn, H), lambda i: (i, 0)),
        ],
        out_shape=[
            jax.ShapeDtypeStruct((H, n, C), jnp.float32),
            jax.ShapeDtypeStruct((n, H), jnp.float32),
            jax.ShapeDtypeStruct((n, H), jnp.float32),
        ],
    )(x, W, att_src_flat, att_dst_flat)


def _tc_edges_body(ea_ref, lw_ref, atte_ref, ae_ref, ael_ref):
    ve_cols = []
    for hd in range(H):
        sl = slice(hd * C, (hd + 1) * C)
        ve_cols.append(jnp.sum(lw_ref[:, sl] * atte_ref[0:1, sl], axis=1, keepdims=True))
    ve = jnp.concatenate(ve_cols, axis=1)
    aeb = jnp.dot(ea_ref[...], ve, preferred_element_type=jnp.float32)
    ae_ref[...] = aeb

    @pl.when(pl.program_id(0) == 0)
    def _():
        ael_ref[...] = jnp.zeros_like(ael_ref)

    ael_ref[...] += jnp.sum(aeb, axis=0)[:, None]


def _tc_edges(edge_attr, lin_edge_W, att_edge_flat):
    E = edge_attr.shape[0]
    be = 512
    grid = (E // be,)
    return pl.pallas_call(
        _tc_edges_body,
        grid=grid,
        in_specs=[
            pl.BlockSpec((be, edge_attr.shape[1]), lambda i: (i, 0)),
            pl.BlockSpec(lin_edge_W.shape, lambda i: (0, 0)),
            pl.BlockSpec((1, H * C), lambda i: (0, 0)),
        ],
        out_specs=[
            pl.BlockSpec((be, H), lambda i: (i, 0)),
            pl.BlockSpec((H, 128), lambda i: (0, 0)),
        ],
        out_shape=[
            jax.ShapeDtypeStruct((E, H), jnp.float32),
            jax.ShapeDtypeStruct((H, 128), jnp.float32),
        ],
    )(edge_attr, lin_edge_W, att_edge_flat)


def _sc_body(n, E, h_hbm, asT_hbm, adT_hbm, aeT_hbm, src_hbm, dst_hbm,
             out_hbm, w_hbm,
             as_v, ad_v,
             srcb0, srcb1, dstb0, dstb1, dsc0, dsc1, gidx0, gidx1,
             aeb0, aeb1, wb0, wb1, hrows0, hrows1, accsh,
             esem0, esem1, gsem0, gsem1, ssem0, ssem1, wsem0, wsem1):
    cid = lax.axis_index("c")
    sid = lax.axis_index("s")
    epw = E // NS
    base_e = sid * epw
    npw = NPAD // NS
    nch = epw // CH

    srcb = (srcb0, srcb1)
    dstb = (dstb0, dstb1)
    dsc = (dsc0, dsc1)
    gidx = (gidx0, gidx1)
    aeb = (aeb0, aeb1)
    wb = (wb0, wb1)
    hrows = (hrows0, hrows1)
    esem = (esem0, esem1)
    gsem = (gsem0, gsem1)
    ssem = (ssem0, ssem1)
    wsem = (wsem0, wsem1)

    for k in range(2):
        head = cid * 2 + k

        pltpu.sync_copy(asT_hbm.at[pl.ds(head * n, n)], as_v)
        pltpu.sync_copy(adT_hbm.at[pl.ds(head * n, n)], ad_v)

        def _z(i, carry):
            r = i // (C // LANES)
            c = (i % (C // LANES)) * LANES
            hrows0[r, pl.ds(c, LANES)] = jnp.zeros((LANES,), jnp.float32)
            return carry

        lax.fori_loop(0, CH * (C // LANES), _z, 0)
        for z in range(npw // CH):
            pltpu.sync_copy(hrows0, accsh.at[pl.ds(sid * npw + z * CH, CH)])
        plsc.subcore_barrier()

        def _estreams(f, eb):
            pltpu.async_copy(src_hbm.at[pl.ds(eb, CH)], srcb[f], esem[f])
            pltpu.async_copy(dst_hbm.at[pl.ds(eb, CH)], dstb[f], esem[f])
            pltpu.async_copy(aeT_hbm.at[pl.ds(head * E + eb, CH)], aeb[f], esem[f])

        _estreams(0, base_e)
        _estreams(1, base_e + CH)

        def _front(f, cf, eb):
            pltpu.make_async_copy(src_hbm.at[pl.ds(eb, CH)], srcb[f], esem[f]).wait()
            pltpu.make_async_copy(dst_hbm.at[pl.ds(eb, CH)], dstb[f], esem[f]).wait()
            pltpu.make_async_copy(aeT_hbm.at[pl.ds(eb, CH)], aeb[f], esem[f]).wait()

            @pl.when(cf >= 2)
            def _():
                pltpu.make_async_copy(wb[f], w_hbm.at[pl.ds(0, CH)], wsem[f]).wait()

            for g in range(CH // LANES):
                s16 = srcb[f][pl.ds(g * LANES, LANES)]
                d16 = dstb[f][pl.ds(g * LANES, LANES)]
                gidx[f][pl.ds(g * LANES, LANES)] = s16 + head * n
                dsc[f][pl.ds(g * LANES, LANES)] = d16
                av = plsc.load_gather(as_v, [s16])
                dv = plsc.load_gather(ad_v, [d16])
                lg = av + dv + aeb[f][pl.ds(g * LANES, LANES)]
                lg = jnp.maximum(lg, 0.2 * lg)
                wb[f][pl.ds(g * LANES, LANES)] = jnp.exp(lg)

            gdesc = pltpu.async_copy(h_hbm.at[gidx[f]], hrows[f], gsem[f])
            pltpu.async_copy(wb[f], w_hbm.at[pl.ds(head * E + eb, CH)], wsem[f])

            @pl.when(cf + 2 < nch)
            def _():
                _estreams(f, eb + 2 * CH)

            return gdesc

        def _back(b, gdesc):
            gdesc.wait()

            def _egrp(g, carry2):
                w16 = wb[b][pl.ds(g * LANES, LANES)]
                for lane in range(LANES):
                    e = g * LANES + lane
                    ws = w16[lane]
                    for j in range(C // LANES):
                        hrows[b][e, pl.ds(j * LANES, LANES)] = (
                            hrows[b][e, pl.ds(j * LANES, LANES)] * ws)
                return carry2

            lax.fori_loop(0, CH // LANES, _egrp, 0)
            sdesc = pltpu.make_async_copy(hrows[b], accsh.at[dsc[b]], ssem[b])
            sdesc.start(add=True)
            return sdesc

        def _pair(p2, carry):
            cf0 = 2 * p2
            cf1 = 2 * p2 + 1
            g0 = _front(0, cf0, base_e + cf0 * CH)
            g1 = _front(1, cf1, base_e + cf1 * CH)
            s0 = _back(0, g0)
            s1 = _back(1, g1)
            s0.wait()
            s1.wait()
            return carry

        lax.fori_loop(0, nch // 2, _pair, 0)

        for f in range(2):
            pltpu.make_async_copy(wb[f], w_hbm.at[pl.ds(0, CH)], wsem[f]).wait()
        plsc.subcore_barrier()

        pltpu.sync_copy(accsh.at[pl.ds(sid * npw, npw)],
                        out_hbm.at[pl.ds(head * NPAD + sid * npw, npw)])
        plsc.subcore_barrier()


def _sc_aggregate(h_flat, asT, adT, aeT, src, dst, n, E):
    mesh = plsc.VectorSubcoreMesh(core_axis_name="c", subcore_axis_name="s",
                                  num_cores=NC, num_subcores=NS)
    fn = pl.kernel(
        functools.partial(_sc_body, n, E),
        out_type=(
            jax.ShapeDtypeStruct((H * NPAD, C), jnp.float32),
            jax.ShapeDtypeStruct((H * E,), jnp.float32),
        ),
        mesh=mesh,
        compiler_params=pltpu.CompilerParams(needs_layout_passes=False),
        scratch_types=[
            pltpu.VMEM((n,), jnp.float32),
            pltpu.VMEM((n,), jnp.float32),
            pltpu.VMEM((CH,), jnp.int32),
            pltpu.VMEM((CH,), jnp.int32),
            pltpu.VMEM((CH,), jnp.int32),
            pltpu.VMEM((CH,), jnp.int32),
            pltpu.VMEM((CH,), jnp.int32),
            pltpu.VMEM((CH,), jnp.int32),
            pltpu.VMEM((CH,), jnp.int32),
            pltpu.VMEM((CH,), jnp.int32),
            pltpu.VMEM((CH,), jnp.float32),
            pltpu.VMEM((CH,), jnp.float32),
            pltpu.VMEM((CH,), jnp.float32),
            pltpu.VMEM((CH,), jnp.float32),
            pltpu.VMEM((CH, C), jnp.float32),
            pltpu.VMEM((CH, C), jnp.float32),
            pltpu.VMEM_SHARED((NPAD, C), jnp.float32),
            pltpu.SemaphoreType.DMA,
            pltpu.SemaphoreType.DMA,
            pltpu.SemaphoreType.DMA,
            pltpu.SemaphoreType.DMA,
            pltpu.SemaphoreType.DMA,
            pltpu.SemaphoreType.DMA,
            pltpu.SemaphoreType.DMA,
            pltpu.SemaphoreType.DMA,
        ],
    )
    return fn(h_flat, asT, adT, aeT, src, dst)


def _tc_s_body(dst_ref, w_ref, s_ref):
    d = dst_ref[...]
    q = lax.shift_right_logical(d, 7)
    r = jnp.bitwise_and(d, 127)
    be = d.shape[0]
    oq = (q == lax.broadcasted_iota(jnp.int32, (be, NPAD // 128), 1)).astype(jnp.float32)
    orr = (r == lax.broadcasted_iota(jnp.int32, (be, 128), 1)).astype(jnp.float32)

    @pl.when(pl.program_id(0) == 0)
    def _():
        s_ref[...] = jnp.zeros_like(s_ref)

    for hd in range(H):
        a = oq * w_ref[:, hd : hd + 1]
        s_ref[hd] += lax.dot_general(a, orr, (((0,), (0,)), ((), ())),
                                     preferred_element_type=jnp.float32)


def _tc_s(dst2d, wT):
    E = dst2d.shape[0]
    be = 512
    grid = (E // be,)
    return pl.pallas_call(
        _tc_s_body,
        grid=grid,
        in_specs=[
            pl.BlockSpec((be, 1), lambda i: (i, 0)),
            pl.BlockSpec((be, H), lambda i: (i, 0)),
        ],
        out_specs=pl.BlockSpec((H, NPAD // 128, 128), lambda i: (0, 0, 0)),
        out_shape=jax.ShapeDtypeStruct((H, NPAD // 128, 128), jnp.float32),
    )(dst2d, wT)


def _tc_final_body(inv_e, acc_ref, h_ref, as_ref, ad_ref, s_ref, ael_ref,
                   bias_ref, out_ref):
    for hd in range(H):
        sl = slice(hd * C, (hd + 1) * C)
        ael = ael_ref[hd : hd + 1, 0:1] * inv_e
        lg = as_ref[:, hd : hd + 1] + ad_ref[:, hd : hd + 1] + ael
        lg = jnp.maximum(lg, 0.2 * lg)
        wl = jnp.exp(lg)
        num = acc_ref[hd] + wl * h_ref[hd]
        den = s_ref[:, hd : hd + 1] + wl
        out_ref[:, sl] = num / den + bias_ref[0:1, sl]


def _tc_final(acc, h, a_src, a_dst, s_nodes, ael, bias_flat, E):
    n = h.shape[1]
    bn = 1000
    grid = (n // bn,)
    return pl.pallas_call(
        functools.partial(_tc_final_body, 1.0 / E),
        grid=grid,
        in_specs=[
            pl.BlockSpec((H, bn, C), lambda i: (0, i, 0)),
            pl.BlockSpec((H, bn, C), lambda i: (0, i, 0)),
            pl.BlockSpec((bn, H), lambda i: (i, 0)),
            pl.BlockSpec((bn, H), lambda i: (i, 0)),
            pl.BlockSpec((bn, H), lambda i: (i, 0)),
            pl.BlockSpec((H, 128), lambda i: (0, 0)),
            pl.BlockSpec((1, H * C), lambda i: (0, 0)),
        ],
        out_specs=pl.BlockSpec((bn, H * C), lambda i: (i, 0)),
        out_shape=jax.ShapeDtypeStruct((n, H * C), jnp.float32),
    )(acc, h, a_src, a_dst, s_nodes, ael, bias_flat)


def kernel(x, edge_index, edge_attr, W, att_src, att_dst, lin_edge_W, att_edge, bias):
    n = x.shape[0]
    E = edge_index.shape[1]
    src = edge_index[0]
    dst = edge_index[1]

    h, a_src, a_dst = _tc_nodes(x, W, att_src.reshape(1, H * C), att_dst.reshape(1, H * C))
    a_e, ael = _tc_edges(edge_attr, lin_edge_W, att_edge.reshape(1, H * C))

    h_flat = h.reshape(H * n, C)
    asT = a_src.T.reshape(H * n)
    adT = a_dst.T.reshape(H * n)
    aeT = a_e.T.reshape(H * E)

    acc, w_planes = _sc_aggregate(h_flat, asT, adT, aeT, src, dst, n, E)

    s = _tc_s(dst.reshape(E, 1), w_planes.reshape(H, E).T)
    s_nodes = s.reshape(H, NPAD).T

    out = _tc_final(acc.reshape(H, NPAD, C), h, a_src, a_dst, s_nodes, ael,
                    bias.reshape(1, H * C), E)
    return out


# speedup vs baseline: 15.7986x; 1.0551x over previous
import functools

import jax
import jax.numpy as jnp
from jax import lax
from jax.experimental import pallas as pl
from jax.experimental.pallas import tpu as pltpu
from jax.experimental.pallas import tpu_sc as plsc

H = 4
C = 128

NC = 2
NS = 16
LANES = 16

CH = 80
NPAD = 10240


def _tc_nodes_body(x_ref, w_ref, asrc_ref, adst_ref, h_ref, as_ref, ad_ref):
    hb = jnp.dot(x_ref[...], w_ref[...], preferred_element_type=jnp.float32)
    as_cols = []
    ad_cols = []
    for hd in range(H):
        sl = slice(hd * C, (hd + 1) * C)
        h_ref[hd] = hb[:, sl]
        as_cols.append(jnp.sum(hb[:, sl] * asrc_ref[0:1, sl], axis=1, keepdims=True))
        ad_cols.append(jnp.sum(hb[:, sl] * adst_ref[0:1, sl], axis=1, keepdims=True))
    as_ref[...] = jnp.concatenate(as_cols, axis=1)
    ad_ref[...] = jnp.concatenate(ad_cols, axis=1)


def _tc_nodes(x, W, att_src_flat, att_dst_flat):
    n = x.shape[0]
    bn = 1000
    grid = (n // bn,)
    return pl.pallas_call(
        _tc_nodes_body,
        grid=grid,
        in_specs=[
            pl.BlockSpec((bn, x.shape[1]), lambda i: (i, 0)),
            pl.BlockSpec(W.shape, lambda i: (0, 0)),
            pl.BlockSpec((1, H * C), lambda i: (0, 0)),
            pl.BlockSpec((1, H * C), lambda i: (0, 0)),
        ],
        out_specs=[
            pl.BlockSpec((H, bn, C), lambda i: (0, i, 0)),
            pl.BlockSpec((bn, H), lambda i: (i, 0)),
            pl.BlockSpec((bn, H), lambda i: (i, 0)),
        ],
        out_shape=[
            jax.ShapeDtypeStruct((H, n, C), jnp.float32),
            jax.ShapeDtypeStruct((n, H), jnp.float32),
            jax.ShapeDtypeStruct((n, H), jnp.float32),
        ],
    )(x, W, att_src_flat, att_dst_flat)


def _tc_edges_body(ea_ref, lw_ref, atte_ref, ae_ref, ael_ref):
    ve_cols = []
    for hd in range(H):
        sl = slice(hd * C, (hd + 1) * C)
        ve_cols.append(jnp.sum(lw_ref[:, sl] * atte_ref[0:1, sl], axis=1, keepdims=True))
    ve = jnp.concatenate(ve_cols, axis=1)
    aeb = jnp.dot(ea_ref[...], ve, preferred_element_type=jnp.float32)
    ae_ref[...] = aeb

    @pl.when(pl.program_id(0) == 0)
    def _():
        ael_ref[...] = jnp.zeros_like(ael_ref)

    ael_ref[...] += jnp.sum(aeb, axis=0)[:, None]


def _tc_edges(edge_attr, lin_edge_W, att_edge_flat):
    E = edge_attr.shape[0]
    be = 512
    grid = (E // be,)
    return pl.pallas_call(
        _tc_edges_body,
        grid=grid,
        in_specs=[
            pl.BlockSpec((be, edge_attr.shape[1]), lambda i: (i, 0)),
            pl.BlockSpec(lin_edge_W.shape, lambda i: (0, 0)),
            pl.BlockSpec((1, H * C), lambda i: (0, 0)),
        ],
        out_specs=[
            pl.BlockSpec((be, H), lambda i: (i, 0)),
            pl.BlockSpec((H, 128), lambda i: (0, 0)),
        ],
        out_shape=[
            jax.ShapeDtypeStruct((E, H), jnp.float32),
            jax.ShapeDtypeStruct((H, 128), jnp.float32),
        ],
    )(edge_attr, lin_edge_W, att_edge_flat)


def _sc_body(n, E, h_hbm, asT_hbm, adT_hbm, aeT_hbm, src_hbm, dst_hbm,
             out_hbm, w_hbm,
             as_v, ad_v,
             srcb0, srcb1, dstb0, dstb1, dsc0, dsc1, gidx0, gidx1,
             aeb0, aeb1, wb0, wb1, hrows0, hrows1, accsh,
             esem0, esem1, gsem0, gsem1, ssem0, ssem1, wsem0, wsem1):
    cid = lax.axis_index("c")
    sid = lax.axis_index("s")
    epw = E // NS
    base_e = sid * epw
    npw = NPAD // NS
    nch = epw // CH

    srcb = (srcb0, srcb1)
    dstb = (dstb0, dstb1)
    dsc = (dsc0, dsc1)
    gidx = (gidx0, gidx1)
    aeb = (aeb0, aeb1)
    wb = (wb0, wb1)
    hrows = (hrows0, hrows1)
    esem = (esem0, esem1)
    gsem = (gsem0, gsem1)
    ssem = (ssem0, ssem1)
    wsem = (wsem0, wsem1)

    for k in range(2):
        head = cid * 2 + k

        pltpu.sync_copy(asT_hbm.at[pl.ds(head * n, n)], as_v)
        pltpu.sync_copy(adT_hbm.at[pl.ds(head * n, n)], ad_v)

        def _z(i, carry):
            r = i // (C // LANES)
            c = (i % (C // LANES)) * LANES
            hrows0[r, pl.ds(c, LANES)] = jnp.zeros((LANES,), jnp.float32)
            return carry

        lax.fori_loop(0, CH * (C // LANES), _z, 0)
        for z in range(npw // CH):
            pltpu.sync_copy(hrows0, accsh.at[pl.ds(sid * npw + z * CH, CH)])
        plsc.subcore_barrier()

        def _estreams(f, eb):
            pltpu.async_copy(src_hbm.at[pl.ds(eb, CH)], srcb[f], esem[f])
            pltpu.async_copy(dst_hbm.at[pl.ds(eb, CH)], dstb[f], esem[f])
            pltpu.async_copy(aeT_hbm.at[pl.ds(head * E + eb, CH)], aeb[f], esem[f])

        _estreams(0, base_e)
        _estreams(1, base_e + CH)

        def _front(f, cf, eb):
            pltpu.make_async_copy(src_hbm.at[pl.ds(eb, CH)], srcb[f], esem[f]).wait()
            pltpu.make_async_copy(dst_hbm.at[pl.ds(eb, CH)], dstb[f], esem[f]).wait()
            pltpu.make_async_copy(aeT_hbm.at[pl.ds(eb, CH)], aeb[f], esem[f]).wait()

            @pl.when(cf >= 2)
            def _():
                pltpu.make_async_copy(wb[f], w_hbm.at[pl.ds(0, CH)], wsem[f]).wait()
                pltpu.make_async_copy(hrows[f], accsh.at[dsc[f]], ssem[f]).wait()

            for g in range(CH // LANES):
                s16 = srcb[f][pl.ds(g * LANES, LANES)]
                d16 = dstb[f][pl.ds(g * LANES, LANES)]
                gidx[f][pl.ds(g * LANES, LANES)] = s16 + head * n
                dsc[f][pl.ds(g * LANES, LANES)] = d16
                av = plsc.load_gather(as_v, [s16])
                dv = plsc.load_gather(ad_v, [d16])
                lg = av + dv + aeb[f][pl.ds(g * LANES, LANES)]
                lg = jnp.maximum(lg, 0.2 * lg)
                wb[f][pl.ds(g * LANES, LANES)] = jnp.exp(lg)

            gdesc = pltpu.async_copy(h_hbm.at[gidx[f]], hrows[f], gsem[f])
            pltpu.async_copy(wb[f], w_hbm.at[pl.ds(head * E + eb, CH)], wsem[f])

            @pl.when(cf + 2 < nch)
            def _():
                _estreams(f, eb + 2 * CH)

            return gdesc

        def _back(b, gdesc):
            gdesc.wait()

            def _egrp(g, carry2):
                w16 = wb[b][pl.ds(g * LANES, LANES)]
                for lane in range(LANES):
                    e = g * LANES + lane
                    ws = w16[lane]
                    for j in range(C // LANES):
                        hrows[b][e, pl.ds(j * LANES, LANES)] = (
                            hrows[b][e, pl.ds(j * LANES, LANES)] * ws)
                return carry2

            lax.fori_loop(0, CH // LANES, _egrp, 0)
            pltpu.make_async_copy(hrows[b], accsh.at[dsc[b]], ssem[b]).start(add=True)

        def _pair(p2, carry):
            cf0 = 2 * p2
            cf1 = 2 * p2 + 1
            g0 = _front(0, cf0, base_e + cf0 * CH)
            g1 = _front(1, cf1, base_e + cf1 * CH)
            _back(0, g0)
            _back(1, g1)
            return carry

        lax.fori_loop(0, nch // 2, _pair, 0)

        for f in range(2):
            pltpu.make_async_copy(wb[f], w_hbm.at[pl.ds(0, CH)], wsem[f]).wait()
            pltpu.make_async_copy(hrows[f], accsh.at[dsc[f]], ssem[f]).wait()
        plsc.subcore_barrier()

        pltpu.sync_copy(accsh.at[pl.ds(sid * npw, npw)],
                        out_hbm.at[pl.ds(head * NPAD + sid * npw, npw)])
        plsc.subcore_barrier()


def _sc_aggregate(h_flat, asT, adT, aeT, src, dst, n, E):
    mesh = plsc.VectorSubcoreMesh(core_axis_name="c", subcore_axis_name="s",
                                  num_cores=NC, num_subcores=NS)
    fn = pl.kernel(
        functools.partial(_sc_body, n, E),
        out_type=(
            jax.ShapeDtypeStruct((H * NPAD, C), jnp.float32),
            jax.ShapeDtypeStruct((H * E,), jnp.float32),
        ),
        mesh=mesh,
        compiler_params=pltpu.CompilerParams(needs_layout_passes=False),
        scratch_types=[
            pltpu.VMEM((n,), jnp.float32),
            pltpu.VMEM((n,), jnp.float32),
            pltpu.VMEM((CH,), jnp.int32),
            pltpu.VMEM((CH,), jnp.int32),
            pltpu.VMEM((CH,), jnp.int32),
            pltpu.VMEM((CH,), jnp.int32),
            pltpu.VMEM((CH,), jnp.int32),
            pltpu.VMEM((CH,), jnp.int32),
            pltpu.VMEM((CH,), jnp.int32),
            pltpu.VMEM((CH,), jnp.int32),
            pltpu.VMEM((CH,), jnp.float32),
            pltpu.VMEM((CH,), jnp.float32),
            pltpu.VMEM((CH,), jnp.float32),
            pltpu.VMEM((CH,), jnp.float32),
            pltpu.VMEM((CH, C), jnp.float32),
            pltpu.VMEM((CH, C), jnp.float32),
            pltpu.VMEM_SHARED((NPAD, C), jnp.float32),
            pltpu.SemaphoreType.DMA,
            pltpu.SemaphoreType.DMA,
            pltpu.SemaphoreType.DMA,
            pltpu.SemaphoreType.DMA,
            pltpu.SemaphoreType.DMA,
            pltpu.SemaphoreType.DMA,
            pltpu.SemaphoreType.DMA,
            pltpu.SemaphoreType.DMA,
        ],
    )
    return fn(h_flat, asT, adT, aeT, src, dst)


def _tc_s_body(dst_ref, w_ref, s_ref):
    d = dst_ref[...]
    q = lax.shift_right_logical(d, 7)
    r = jnp.bitwise_and(d, 127)
    be = d.shape[0]
    oq = (q == lax.broadcasted_iota(jnp.int32, (be, NPAD // 128), 1)).astype(jnp.float32)
    orr = (r == lax.broadcasted_iota(jnp.int32, (be, 128), 1)).astype(jnp.float32)

    @pl.when(pl.program_id(0) == 0)
    def _():
        s_ref[...] = jnp.zeros_like(s_ref)

    for hd in range(H):
        a = oq * w_ref[:, hd : hd + 1]
        s_ref[hd] += lax.dot_general(a, orr, (((0,), (0,)), ((), ())),
                                     preferred_element_type=jnp.float32)


def _tc_s(dst2d, wT):
    E = dst2d.shape[0]
    be = 512
    grid = (E // be,)
    return pl.pallas_call(
        _tc_s_body,
        grid=grid,
        in_specs=[
            pl.BlockSpec((be, 1), lambda i: (i, 0)),
            pl.BlockSpec((be, H), lambda i: (i, 0)),
        ],
        out_specs=pl.BlockSpec((H, NPAD // 128, 128), lambda i: (0, 0, 0)),
        out_shape=jax.ShapeDtypeStruct((H, NPAD // 128, 128), jnp.float32),
    )(dst2d, wT)


def _tc_final_body(inv_e, acc_ref, h_ref, as_ref, ad_ref, s_ref, ael_ref,
                   bias_ref, out_ref):
    for hd in range(H):
        sl = slice(hd * C, (hd + 1) * C)
        ael = ael_ref[hd : hd + 1, 0:1] * inv_e
        lg = as_ref[:, hd : hd + 1] + ad_ref[:, hd : hd + 1] + ael
        lg = jnp.maximum(lg, 0.2 * lg)
        wl = jnp.exp(lg)
        num = acc_ref[hd] + wl * h_ref[hd]
        den = s_ref[:, hd : hd + 1] + wl
        out_ref[:, sl] = num / den + bias_ref[0:1, sl]


def _tc_final(acc, h, a_src, a_dst, s_nodes, ael, bias_flat, E):
    n = h.shape[1]
    bn = 1000
    grid = (n // bn,)
    return pl.pallas_call(
        functools.partial(_tc_final_body, 1.0 / E),
        grid=grid,
        in_specs=[
            pl.BlockSpec((H, bn, C), lambda i: (0, i, 0)),
            pl.BlockSpec((H, bn, C), lambda i: (0, i, 0)),
            pl.BlockSpec((bn, H), lambda i: (i, 0)),
            pl.BlockSpec((bn, H), lambda i: (i, 0)),
            pl.BlockSpec((bn, H), lambda i: (i, 0)),
            pl.BlockSpec((H, 128), lambda i: (0, 0)),
            pl.BlockSpec((1, H * C), lambda i: (0, 0)),
        ],
        out_specs=pl.BlockSpec((bn, H * C), lambda i: (i, 0)),
        out_shape=jax.ShapeDtypeStruct((n, H * C), jnp.float32),
    )(acc, h, a_src, a_dst, s_nodes, ael, bias_flat)


def kernel(x, edge_index, edge_attr, W, att_src, att_dst, lin_edge_W, att_edge, bias):
    n = x.shape[0]
    E = edge_index.shape[1]
    src = edge_index[0]
    dst = edge_index[1]

    h, a_src, a_dst = _tc_nodes(x, W, att_src.reshape(1, H * C), att_dst.reshape(1, H * C))
    a_e, ael = _tc_edges(edge_attr, lin_edge_W, att_edge.reshape(1, H * C))

    h_flat = h.reshape(H * n, C)
    asT = a_src.T.reshape(H * n)
    adT = a_dst.T.reshape(H * n)
    aeT = a_e.T.reshape(H * E)

    acc, w_planes = _sc_aggregate(h_flat, asT, adT, aeT, src, dst, n, E)

    s = _tc_s(dst.reshape(E, 1), w_planes.reshape(H, E).T)
    s_nodes = s.reshape(H, NPAD).T

    out = _tc_final(acc.reshape(H, NPAD, C), h, a_src, a_dst, s_nodes, ael,
                    bias.reshape(1, H * C), E)
    return out
